# baseline (speedup 1.0000x reference)
import functools

import jax
import jax.numpy as jnp
from jax import lax
from jax.experimental import pallas as pl
from jax.experimental.pallas import tpu as pltpu
from jax.experimental.pallas import tpu_sc as plsc

N_NODES = 10000
N_EDGES = 320000
D_FEAT = 128

NC = 2
NS = 16
NW = NC * NS
E_TILE = N_EDGES // NW
CHUNK = 80
CHUNKS = E_TILE // CHUNK
SUB = CHUNK // 16
ROWS_TILE = N_NODES // NS
NBUF = 4

_mesh = plsc.VectorSubcoreMesh(
    core_axis_name="c", subcore_axis_name="s", num_cores=NC, num_subcores=NS
)


@functools.partial(
    pl.kernel,
    out_type=jax.ShapeDtypeStruct((NC, N_NODES, D_FEAT), jnp.float32),
    mesh=_mesh,
    compiler_params=pltpu.CompilerParams(
        use_tc_tiling_on_sc=False, needs_layout_passes=False
    ),
    scratch_types=[
        [pltpu.VMEM((3, CHUNK), jnp.int32) for _ in range(NBUF)],
        [pltpu.VMEM((CHUNK, D_FEAT), jnp.float32) for _ in range(NBUF)],
        pltpu.VMEM_SHARED((N_NODES, D_FEAT), jnp.float32),
        [pltpu.SemaphoreType.DMA for _ in range(NBUF)],
        [pltpu.SemaphoreType.DMA for _ in range(NBUF)],
        [pltpu.SemaphoreType.DMA for _ in range(NBUF)],
    ],
)
def _sc_scatter(x_hbm, meta_hbm, zero_hbm, out_hbm,
                meta_v, rows_v, acc_sh, msem, gsem, ssem):
    cid = lax.axis_index("c")
    sid = lax.axis_index("s")
    wid = sid * NC + cid

    pltpu.sync_copy(zero_hbm, acc_sh.at[pl.ds(sid * ROWS_TILE, ROWS_TILE)])
    plsc.subcore_barrier()

    def _m_start(j, b):
        pltpu.async_copy(meta_hbm.at[wid, j], meta_v[b], msem[b])

    def _m_wait(j, b):
        pltpu.make_async_copy(meta_hbm.at[wid, j], meta_v[b], msem[b]).wait()

    def _g_start(b):
        pltpu.async_copy(x_hbm.at[meta_v[b].at[0]], rows_v[b], gsem[b])

    def _g_wait(b):
        pltpu.make_async_copy(
            x_hbm.at[meta_v[b].at[0]], rows_v[b], gsem[b]
        ).wait()

    def _s_start(b):
        pltpu.async_copy(
            rows_v[b], acc_sh.at[meta_v[b].at[1]], ssem[b], add=True
        )

    def _s_wait(b):
        pltpu.make_async_copy(
            rows_v[b], acc_sh.at[meta_v[b].at[1]], ssem[b]
        ).wait()

    def _scale(b):
        buf = rows_v[b]

        def _group(s, c2):
            vbits = meta_v[b][2, pl.ds(s * 16, 16)]
            val16 = plsc.bitcast(vbits, jnp.float32)
            for e16 in range(16):
                sv = jnp.take_along_axis(
                    val16, jnp.full((16,), e16, jnp.int32), axis=0
                )
                e = s * 16 + e16
                for k in range(D_FEAT // 16):
                    buf[e, pl.ds(k * 16, 16)] = (
                        buf[e, pl.ds(k * 16, 16)] * sv
                    )
            return c2

        lax.fori_loop(0, SUB, _group, 0)

    _m_start(0, 0)
    _m_wait(0, 0)
    _g_start(0)

    def _iter(i, carry):
        for u in range(NBUF):
            j = NBUF * i + u

            @pl.when(j >= NBUF - 1)
            def _():
                _s_wait((u + 1) % NBUF)

            _m_start(j + 1, (u + 1) % NBUF)
            _g_wait(u)
            _scale(u)
            _m_wait(j + 1, (u + 1) % NBUF)
            _g_start((u + 1) % NBUF)
            _s_start(u)
        return carry

    lax.fori_loop(0, (CHUNKS - 1) // NBUF, _iter, 0)

    _s_wait(1)
    _g_wait(0)
    _scale(0)
    _s_start(0)
    _s_wait(2)
    _s_wait(3)
    _s_wait(0)
    plsc.subcore_barrier()

    pltpu.sync_copy(
        acc_sh.at[pl.ds(sid * ROWS_TILE, ROWS_TILE)],
        out_hbm.at[cid, pl.ds(sid * ROWS_TILE, ROWS_TILE)],
    )


def _combine_body(a_ref, b_ref, o_ref):
    o_ref[...] = a_ref[...] + b_ref[...]


_combine = pl.pallas_call(
    _combine_body,
    out_shape=jax.ShapeDtypeStruct((N_NODES, D_FEAT), jnp.float32),
    grid=(5,),
    in_specs=[
        pl.BlockSpec((2000, D_FEAT), lambda i: (i, 0)),
        pl.BlockSpec((2000, D_FEAT), lambda i: (i, 0)),
    ],
    out_specs=pl.BlockSpec((2000, D_FEAT), lambda i: (i, 0)),
)


@jax.jit
def kernel(x_source, edge_index, values):
    src = edge_index[1].reshape(NW, CHUNKS, 1, CHUNK)
    tgt = edge_index[0].reshape(NW, CHUNKS, 1, CHUNK)
    val = lax.bitcast_convert_type(values, jnp.int32).reshape(
        NW, CHUNKS, 1, CHUNK
    )
    meta = jnp.concatenate([src, tgt, val], axis=2)
    zero = jnp.zeros((ROWS_TILE, D_FEAT), jnp.float32)
    partial = _sc_scatter(x_source, meta, zero)
    return _combine(partial[0], partial[1])
